# Initial kernel scaffold; baseline (speedup 1.0000x reference)
import jax
import jax.numpy as jnp
from jax.experimental import pallas as pl


def kernel(x, et, H, raw_edge_index, lg_edge_index, W_tsa_in, a_src, a_dst, W_tsa_v, W_etn, W_egcn, W_ea_self, W_ea_neigh, W_an1_self, W_an1_neigh, W_an2_self, W_an2_neigh, W_mix_n, W_mix_e, a_mix, W_out):
    raise NotImplementedError("write your pallas kernel here")


# speedup vs baseline: 12.1704x; 12.1704x over previous
import functools

import jax
import jax.numpy as jnp
from jax import lax
from jax.experimental import pallas as pl
from jax.experimental.pallas import tpu as pltpu, tpu_sc as plsc

N = 10000
E = 320000
ELG = 640000
F = 128
T = 16
HID = 128
OUT = 64

NC = 2
NS = 16
NW = NC * NS

ELG_P = 655360
E_P = 327680
G_P = 327680

CH2 = 2048
EACC = 320256
CH3 = 1280
GROWS = 81920
GACC = 81928
CH5 = 80
CH5B = 2000
NACC1 = 10240
CH7 = 256
NACC2 = 10016

f32 = jnp.float32
i32 = jnp.int32

def _wr_nrows(acc, out, s, nrw):
    @pl.when(nrw == 624)
    def _():
        pltpu.sync_copy(acc.at[pl.ds(s * 624, 624)], out.at[pl.ds(s * 624, 624)])
    @pl.when(nrw == 640)
    def _():
        pltpu.sync_copy(acc.at[pl.ds(s * 624, 640)], out.at[pl.ds(s * 624, 640)])


_mesh = plsc.VectorSubcoreMesh(core_axis_name="c", subcore_axis_name="s")
_sc_packed = pltpu.CompilerParams(use_tc_tiling_on_sc=False)


@functools.partial(
    pl.kernel,
    out_type=(
        jax.ShapeDtypeStruct((ELG_P,), f32),
        jax.ShapeDtypeStruct((E,), f32),
        jax.ShapeDtypeStruct((E,), f32),
    ),
    mesh=_mesh,
    compiler_params=_sc_packed,
    scratch_types=dict(
        ls_v=pltpu.VMEM((CH2,), i32),
        ld_v=pltpu.VMEM((CH2,), i32),
        g1_v=pltpu.VMEM((CH2,), f32),
        g2_v=pltpu.VMEM((CH2,), f32),
        ex_v=pltpu.VMEM((CH2,), f32),
        z_v=pltpu.VMEM((CH2,), f32),
        acc=pltpu.VMEM_SHARED((EACC,), f32),
        sem=pltpu.SemaphoreType.DMA,
    ),
)
def sc_scores(s1, s2p, lsrc, ldst, ex_out, d0, d1,
              ls_v, ld_v, g1_v, g2_v, ex_v, z_v, acc, sem):
    c = lax.axis_index("c")
    s = lax.axis_index("s")
    wid = s * NC + c

    def zb(i, _):
        z_v[pl.ds(i * 16, 16)] = jnp.zeros((16,), f32)
        return 0
    lax.fori_loop(0, CH2 // 16, zb, 0)
    zbase = s * 20016
    for k in range(9):
        pltpu.sync_copy(z_v, acc.at[pl.ds(zbase + k * CH2, CH2)])
    pltpu.sync_copy(z_v.at[pl.ds(0, 1584)], acc.at[pl.ds(zbase + 9 * CH2, 1584)])
    plsc.subcore_barrier()

    base = wid * (ELG_P // NW)

    def chunk(k, _):
        off = base + k * CH2
        pltpu.sync_copy(lsrc.at[pl.ds(off, CH2)], ls_v)
        pltpu.sync_copy(ldst.at[pl.ds(off, CH2)], ld_v)
        pltpu.async_copy(s1.at[ls_v], g1_v, sem).wait()
        pltpu.async_copy(s2p.at[ld_v], g2_v, sem).wait()

        def grp(g, _):
            v = g1_v[pl.ds(g * 16, 16)] + g2_v[pl.ds(g * 16, 16)]
            v = jnp.where(v >= 0, v, 0.2 * v)
            ex_v[pl.ds(g * 16, 16)] = jnp.exp(v)
            return 0
        lax.fori_loop(0, CH2 // 16, grp, 0)

        pltpu.sync_copy(ex_v, ex_out.at[pl.ds(off, CH2)])
        pltpu.sync_copy(ex_v, acc.at[ld_v], add=True)
        return 0
    lax.fori_loop(0, (ELG_P // NW) // CH2, chunk, 0)
    plsc.subcore_barrier()

    wbase = s * (E // NS)
    @pl.when(c == 0)
    def _():
        pltpu.sync_copy(acc.at[pl.ds(wbase, E // NS)], d0.at[pl.ds(wbase, E // NS)])
    @pl.when(c == 1)
    def _():
        pltpu.sync_copy(acc.at[pl.ds(wbase, E // NS)], d1.at[pl.ds(wbase, E // NS)])


@functools.partial(
    pl.kernel,
    out_type=jax.ShapeDtypeStruct((G_P, T), f32),
    mesh=_mesh,
    compiler_params=_sc_packed,
    scratch_types=dict(
        ls_v=pltpu.VMEM((CH3,), i32),
        ld_v=pltpu.VMEM((CH3,), i32),
        li_v=pltpu.VMEM((CH3,), i32),
        ex_v=pltpu.VMEM((CH3,), f32),
        s_v=pltpu.VMEM((CH3, T), f32),
        z_v=pltpu.VMEM((512, T), f32),
        acc=pltpu.VMEM_SHARED((GACC, T), f32),
        sem=pltpu.SemaphoreType.DMA,
    ),
)
def sc_gacc(et, lsrc, ldst, ex, g_out,
            ls_v, ld_v, li_v, ex_v, s_v, z_v, acc, sem):
    c = lax.axis_index("c")
    s = lax.axis_index("s")

    def zrow(i, _):
        z_v[i, :] = jnp.zeros((T,), f32)
        return 0
    lax.fori_loop(0, 512, zrow, 0)

    for r in range(2):
        lo = jnp.where(c == 0, r * 2 * GROWS, (r * 2 + 1) * GROWS).astype(i32)
        hi = lo + GROWS
        zb = s * (GROWS // NS)
        for k in range(GROWS // NS // 512):
            pltpu.sync_copy(z_v, acc.at[pl.ds(zb + k * 512, 512)])
        plsc.subcore_barrier()

        base = s * (ELG_P // NS)

        def chunk(k, _):
            off = base + k * CH3
            pltpu.sync_copy(lsrc.at[pl.ds(off, CH3)], ls_v)
            pltpu.sync_copy(ldst.at[pl.ds(off, CH3)], ld_v)
            pltpu.sync_copy(ex.at[pl.ds(off, CH3)], ex_v)
            pltpu.async_copy(et.at[ls_v], s_v, sem).wait()

            def grp(g, _):
                ldg = ld_v[pl.ds(g * 16, 16)]
                inr = (ldg >= lo) & (ldg < hi)
                li_v[pl.ds(g * 16, 16)] = jnp.where(inr, ldg - lo, GROWS)
                exg = ex_v[pl.ds(g * 16, 16)]
                for j in range(16):
                    row = g * 16 + j
                    s_v[row, :] = s_v[row, :] * exg[j]
                return 0
            lax.fori_loop(0, CH3 // 16, grp, 0)

            pltpu.sync_copy(s_v, acc.at[li_v], add=True)
            return 0
        lax.fori_loop(0, (ELG_P // NS) // CH3, chunk, 0)
        plsc.subcore_barrier()

        rps = GROWS // NS
        pltpu.sync_copy(acc.at[pl.ds(s * rps, rps)],
                        g_out.at[pl.ds(lo + s * rps, rps)])
        plsc.subcore_barrier()


@functools.partial(
    pl.kernel,
    out_type=(
        jax.ShapeDtypeStruct((N, HID), f32),
        jax.ShapeDtypeStruct((N, HID), f32),
    ),
    mesh=_mesh,
    scratch_types=dict(
        h_v=pltpu.VMEM((CH5,), i32),
        t_v=pltpu.VMEM((CH5, HID), f32),
        z_v=pltpu.VMEM((64, HID), f32),
        acc=pltpu.VMEM_SHARED((N, HID), f32),
        sem=pltpu.SemaphoreType.DMA,
    ),
)
def sc_nsum(tsae, h_idx, p0, p1, h_v, t_v, z_v, acc, sem):
    c = lax.axis_index("c")
    s = lax.axis_index("s")
    wid = s * NC + c

    def zrow(i, _):
        for j in range(HID // 16):
            z_v[i, pl.ds(j * 16, 16)] = jnp.zeros((16,), f32)
        return 0
    lax.fori_loop(0, 64, zrow, 0)
    zb = s * (N // NS)
    for k in range(9):
        pltpu.sync_copy(z_v, acc.at[pl.ds(zb + k * 64, 64)])
    pltpu.sync_copy(z_v.at[pl.ds(0, 49)], acc.at[pl.ds(zb + 576, 49)])
    plsc.subcore_barrier()

    base = wid * (E // NW)

    def chunk(k, _):
        off = base + k * CH5
        pltpu.sync_copy(h_idx.at[pl.ds(off, CH5)], h_v)
        pltpu.sync_copy(tsae.at[pl.ds(off, CH5), :], t_v)
        pltpu.sync_copy(t_v, acc.at[h_v], add=True)
        return 0
    lax.fori_loop(0, (E // NW) // CH5, chunk, 0)
    plsc.subcore_barrier()

    nrw = jnp.where(s == NS - 1, 640, 624).astype(i32)
    @pl.when(c == 0)
    def _():
        _wr_nrows(acc, p0, s, nrw)
    @pl.when(c == 1)
    def _():
        _wr_nrows(acc, p1, s, nrw)


@functools.partial(
    pl.kernel,
    out_type=(
        jax.ShapeDtypeStruct((N,), f32),
        jax.ShapeDtypeStruct((N,), f32),
    ),
    mesh=_mesh,
    compiler_params=_sc_packed,
    scratch_types=dict(
        i_v=pltpu.VMEM((CH5B,), i32),
        one_v=pltpu.VMEM((CH5B,), f32),
        z_v=pltpu.VMEM((640,), f32),
        acc=pltpu.VMEM_SHARED((NACC1,), f32),
        sem=pltpu.SemaphoreType.DMA,
    ),
)
def sc_counts(h_idx, rdst, cnt_h, cnt_r, i_v, one_v, z_v, acc, sem):
    c = lax.axis_index("c")
    s = lax.axis_index("s")

    def ob(i, _):
        one_v[pl.ds(i * 16, 16)] = jnp.ones((16,), f32)
        return 0
    lax.fori_loop(0, CH5B // 16, ob, 0)
    def zb(i, _):
        z_v[pl.ds(i * 16, 16)] = jnp.zeros((16,), f32)
        return 0
    lax.fori_loop(0, 40, zb, 0)
    pltpu.sync_copy(z_v, acc.at[pl.ds(s * 640, 640)])
    plsc.subcore_barrier()

    base = s * (E // NS)

    def chunk_src(src):
        def chunk(k, _):
            off = base + k * CH5B
            pltpu.sync_copy(src.at[pl.ds(off, CH5B)], i_v)
            pltpu.sync_copy(one_v, acc.at[i_v], add=True)
            return 0
        return chunk

    @pl.when(c == 0)
    def _():
        lax.fori_loop(0, (E // NS) // CH5B, chunk_src(h_idx), 0)
    @pl.when(c == 1)
    def _():
        lax.fori_loop(0, (E // NS) // CH5B, chunk_src(rdst), 0)
    plsc.subcore_barrier()

    @pl.when(s == 0)
    def _():
        @pl.when(c == 0)
        def _():
            pltpu.sync_copy(acc.at[pl.ds(0, N)], cnt_h)
        @pl.when(c == 1)
        def _():
            pltpu.sync_copy(acc.at[pl.ds(0, N)], cnt_r)


@functools.partial(
    pl.kernel,
    out_type=(
        jax.ShapeDtypeStruct((N, HID), f32),
        jax.ShapeDtypeStruct((N, HID), f32),
    ),
    mesh=_mesh,
    scratch_types=dict(
        si_v=pltpu.VMEM((CH7,), i32),
        di_v=pltpu.VMEM((CH7,), i32),
        r_v=pltpu.VMEM((CH7, HID), f32),
        z_v=pltpu.VMEM((64, HID), f32),
        acc=pltpu.VMEM_SHARED((NACC2, HID), f32),
        sem=pltpu.SemaphoreType.DMA,
    ),
)
def sc_agg2(xt, ert, rsrc, rdst, aggx, agger, si_v, di_v, r_v, z_v, acc, sem):
    c = lax.axis_index("c")
    s = lax.axis_index("s")

    def zrow(i, _):
        for j in range(HID // 16):
            z_v[i, pl.ds(j * 16, 16)] = jnp.zeros((16,), f32)
        return 0
    lax.fori_loop(0, 64, zrow, 0)
    zb = s * (NACC2 // NS)
    for k in range(9):
        pltpu.sync_copy(z_v, acc.at[pl.ds(zb + k * 64, 64)])
    pltpu.sync_copy(z_v.at[pl.ds(0, 50)], acc.at[pl.ds(zb + 576, 50)])
    plsc.subcore_barrier()

    base = s * (E_P // NS)

    def chunk_tab(tab):
        def chunk(k, _):
            off = base + k * CH7
            pltpu.sync_copy(rsrc.at[pl.ds(off, CH7)], si_v)
            pltpu.sync_copy(rdst.at[pl.ds(off, CH7)], di_v)
            pltpu.async_copy(tab.at[si_v], r_v, sem).wait()
            pltpu.sync_copy(r_v, acc.at[di_v], add=True)
            return 0
        return chunk

    @pl.when(c == 0)
    def _():
        lax.fori_loop(0, (E_P // NS) // CH7, chunk_tab(xt), 0)
    @pl.when(c == 1)
    def _():
        lax.fori_loop(0, (E_P // NS) // CH7, chunk_tab(ert), 0)
    plsc.subcore_barrier()

    nrw = jnp.where(s == NS - 1, 640, 624).astype(i32)
    @pl.when(c == 0)
    def _():
        _wr_nrows(acc, aggx, s, nrw)
    @pl.when(c == 1)
    def _():
        _wr_nrows(acc, agger, s, nrw)


@functools.partial(
    pl.kernel,
    out_type=(
        jax.ShapeDtypeStruct((N, HID), f32),
        jax.ShapeDtypeStruct((N, HID), f32),
    ),
    mesh=_mesh,
    scratch_types=dict(
        si_v=pltpu.VMEM((CH7,), i32),
        di_v=pltpu.VMEM((CH7,), i32),
        r_v=pltpu.VMEM((CH7, HID), f32),
        z_v=pltpu.VMEM((64, HID), f32),
        acc=pltpu.VMEM_SHARED((NACC2, HID), f32),
        sem=pltpu.SemaphoreType.DMA,
    ),
)
def sc_agg1(tab, rsrc, rdst, p0, p1, si_v, di_v, r_v, z_v, acc, sem):
    c = lax.axis_index("c")
    s = lax.axis_index("s")
    wid = s * NC + c

    def zrow(i, _):
        for j in range(HID // 16):
            z_v[i, pl.ds(j * 16, 16)] = jnp.zeros((16,), f32)
        return 0
    lax.fori_loop(0, 64, zrow, 0)
    zb = s * (NACC2 // NS)
    for k in range(9):
        pltpu.sync_copy(z_v, acc.at[pl.ds(zb + k * 64, 64)])
    pltpu.sync_copy(z_v.at[pl.ds(0, 50)], acc.at[pl.ds(zb + 576, 50)])
    plsc.subcore_barrier()

    base = wid * (E_P // NW)

    def chunk(k, _):
        off = base + k * CH7
        pltpu.sync_copy(rsrc.at[pl.ds(off, CH7)], si_v)
        pltpu.sync_copy(rdst.at[pl.ds(off, CH7)], di_v)
        pltpu.async_copy(tab.at[si_v], r_v, sem).wait()
        pltpu.sync_copy(r_v, acc.at[di_v], add=True)
        return 0
    lax.fori_loop(0, (E_P // NW) // CH7, chunk, 0)
    plsc.subcore_barrier()

    nrw = jnp.where(s == NS - 1, 640, 624).astype(i32)
    @pl.when(c == 0)
    def _():
        _wr_nrows(acc, p0, s, nrw)
    @pl.when(c == 1)
    def _():
        _wr_nrows(acc, p1, s, nrw)


BLK_E = 3200
BLK_N = 1000


def _tc_prep_body(et_ref, w_ref, a1_ref, a2_ref, s1_ref, s2_ref):
    h = jnp.dot(et_ref[...], w_ref[...], preferred_element_type=f32)
    s1_ref[...] = jnp.dot(h, a1_ref[...], preferred_element_type=f32)
    s2_ref[...] = jnp.dot(h, a2_ref[...], preferred_element_type=f32)


def tc_prep(et, w_tsa_in, a_src, a_dst):
    nb = E // BLK_E
    return pl.pallas_call(
        _tc_prep_body,
        grid=(nb,),
        in_specs=[
            pl.BlockSpec((BLK_E, T), lambda i: (i, 0)),
            pl.BlockSpec((T, HID), lambda i: (0, 0)),
            pl.BlockSpec((HID, 1), lambda i: (0, 0)),
            pl.BlockSpec((HID, 1), lambda i: (0, 0)),
        ],
        out_specs=[
            pl.BlockSpec((BLK_E, 1), lambda i: (i, 0)),
            pl.BlockSpec((BLK_E, 1), lambda i: (i, 0)),
        ],
        out_shape=[
            jax.ShapeDtypeStruct((E, 1), f32),
            jax.ShapeDtypeStruct((E, 1), f32),
        ],
    )(et, w_tsa_in, a_src, a_dst)


def _tc_tsae_body(et_ref, g_ref, d0_ref, d1_ref, w1_ref, wv_ref, o_ref):
    wc = jnp.dot(w1_ref[...], wv_ref[...], preferred_element_type=f32)
    inv = 1.0 / (d0_ref[...] + d1_ref[...] + 1e-16)
    h = jnp.dot(et_ref[...], w1_ref[...], preferred_element_type=f32)
    agg = jnp.dot(g_ref[...] * inv, wc, preferred_element_type=f32)
    o_ref[...] = jnp.maximum(h + agg, 0.0)


def tc_tsae(et, g, d0, d1, w_tsa_in, w_tsa_v):
    nb = E // BLK_E
    return pl.pallas_call(
        _tc_tsae_body,
        grid=(nb,),
        in_specs=[
            pl.BlockSpec((BLK_E, T), lambda i: (i, 0)),
            pl.BlockSpec((BLK_E, T), lambda i: (i, 0)),
            pl.BlockSpec((BLK_E, 1), lambda i: (i, 0)),
            pl.BlockSpec((BLK_E, 1), lambda i: (i, 0)),
            pl.BlockSpec((T, HID), lambda i: (0, 0)),
            pl.BlockSpec((HID, HID), lambda i: (0, 0)),
        ],
        out_specs=pl.BlockSpec((BLK_E, HID), lambda i: (i, 0)),
        out_shape=jax.ShapeDtypeStruct((E, HID), f32),
    )(et, g, d0, d1, w_tsa_in, w_tsa_v)


def _tc_er_body(p0_ref, p1_ref, cnt_ref, wet_ref, weg_ref, o_ref):
    inv = 1.0 / jnp.maximum(cnt_ref[...], 1.0)
    mean = (p0_ref[...] + p1_ref[...]) * inv
    etn = jnp.dot(mean, wet_ref[...], preferred_element_type=f32)
    lre = jnp.where(etn >= 0, etn, 0.2 * etn)
    o_ref[...] = jnp.dot(lre, weg_ref[...], preferred_element_type=f32)


def tc_edge_repr(p0, p1, cnt_h, w_etn, w_egcn):
    nb = N // BLK_N
    return pl.pallas_call(
        _tc_er_body,
        grid=(nb,),
        in_specs=[
            pl.BlockSpec((BLK_N, HID), lambda i: (i, 0)),
            pl.BlockSpec((BLK_N, HID), lambda i: (i, 0)),
            pl.BlockSpec((BLK_N, 1), lambda i: (i, 0)),
            pl.BlockSpec((HID, HID), lambda i: (0, 0)),
            pl.BlockSpec((HID, HID), lambda i: (0, 0)),
        ],
        out_specs=pl.BlockSpec((BLK_N, HID), lambda i: (i, 0)),
        out_shape=jax.ShapeDtypeStruct((N, HID), f32),
    )(p0, p1, cnt_h, w_etn, w_egcn)


def _tc_sage1_body(x_ref, er_ref, ax_ref, ae_ref, cnt_ref,
                   ws1_ref, wn1_ref, wes_ref, wen_ref, nh_ref, aer_ref):
    inv = 1.0 / jnp.maximum(cnt_ref[...], 1.0)
    nh = (jnp.dot(x_ref[...], ws1_ref[...], preferred_element_type=f32)
          + jnp.dot(ax_ref[...] * inv, wn1_ref[...], preferred_element_type=f32))
    nh_ref[...] = jnp.maximum(nh, 0.0)
    aer_ref[...] = (jnp.dot(er_ref[...], wes_ref[...], preferred_element_type=f32)
                    + jnp.dot(ae_ref[...] * inv, wen_ref[...], preferred_element_type=f32))


def tc_sage1(x, er, aggx, agger, cnt_r, w_an1s, w_an1n, w_eas, w_ean):
    nb = N // BLK_N
    return pl.pallas_call(
        _tc_sage1_body,
        grid=(nb,),
        in_specs=[
            pl.BlockSpec((BLK_N, F), lambda i: (i, 0)),
            pl.BlockSpec((BLK_N, HID), lambda i: (i, 0)),
            pl.BlockSpec((BLK_N, F), lambda i: (i, 0)),
            pl.BlockSpec((BLK_N, HID), lambda i: (i, 0)),
            pl.BlockSpec((BLK_N, 1), lambda i: (i, 0)),
            pl.BlockSpec((F, HID), lambda i: (0, 0)),
            pl.BlockSpec((F, HID), lambda i: (0, 0)),
            pl.BlockSpec((HID, HID), lambda i: (0, 0)),
            pl.BlockSpec((HID, HID), lambda i: (0, 0)),
        ],
        out_specs=[
            pl.BlockSpec((BLK_N, HID), lambda i: (i, 0)),
            pl.BlockSpec((BLK_N, HID), lambda i: (i, 0)),
        ],
        out_shape=[
            jax.ShapeDtypeStruct((N, HID), f32),
            jax.ShapeDtypeStruct((N, HID), f32),
        ],
    )(x, er, aggx, agger, cnt_r, w_an1s, w_an1n, w_eas, w_ean)


def _tc_final_body(nh_ref, aer_ref, p0_ref, p1_ref, cnt_ref,
                   w2s_ref, w2n_ref, wmn_ref, wme_ref, am_ref, wo_ref, o_ref):
    inv = 1.0 / jnp.maximum(cnt_ref[...], 1.0)
    nr = (jnp.dot(nh_ref[...], w2s_ref[...], preferred_element_type=f32)
          + jnp.dot((p0_ref[...] + p1_ref[...]) * inv, w2n_ref[...],
                    preferred_element_type=f32))
    zn = jnp.dot(nr, wmn_ref[...], preferred_element_type=f32)
    ze = jnp.dot(aer_ref[...], wme_ref[...], preferred_element_type=f32)
    am = am_ref[...]
    gs = (jnp.sum(zn * am[0:1, :], axis=1, keepdims=True)
          + jnp.sum(ze * am[1:2, :], axis=1, keepdims=True))
    gate = jax.nn.sigmoid(gs)
    mixed = gate * zn + (1.0 - gate) * ze
    logits = jnp.dot(mixed, wo_ref[...], preferred_element_type=f32)
    mx = jnp.max(logits, axis=1, keepdims=True)
    lse = mx + jnp.log(jnp.sum(jnp.exp(logits - mx), axis=1, keepdims=True))
    o_ref[...] = logits - lse


def tc_final(nh, aer, p0, p1, cnt_r, w2s, w2n, wmn, wme, am2, wo):
    nb = N // BLK_N
    return pl.pallas_call(
        _tc_final_body,
        grid=(nb,),
        in_specs=[
            pl.BlockSpec((BLK_N, HID), lambda i: (i, 0)),
            pl.BlockSpec((BLK_N, HID), lambda i: (i, 0)),
            pl.BlockSpec((BLK_N, HID), lambda i: (i, 0)),
            pl.BlockSpec((BLK_N, HID), lambda i: (i, 0)),
            pl.BlockSpec((BLK_N, 1), lambda i: (i, 0)),
            pl.BlockSpec((HID, HID), lambda i: (0, 0)),
            pl.BlockSpec((HID, HID), lambda i: (0, 0)),
            pl.BlockSpec((HID, HID), lambda i: (0, 0)),
            pl.BlockSpec((HID, HID), lambda i: (0, 0)),
            pl.BlockSpec((2, HID), lambda i: (0, 0)),
            pl.BlockSpec((HID, OUT), lambda i: (0, 0)),
        ],
        out_specs=pl.BlockSpec((BLK_N, OUT), lambda i: (i, 0)),
        out_shape=jax.ShapeDtypeStruct((N, OUT), f32),
    )(nh, aer, p0, p1, cnt_r, w2s, w2n, wmn, wme, am2, wo)


def kernel(x, et, H, raw_edge_index, lg_edge_index, W_tsa_in, a_src, a_dst,
           W_tsa_v, W_etn, W_egcn, W_ea_self, W_ea_neigh, W_an1_self,
           W_an1_neigh, W_an2_self, W_an2_neigh, W_mix_n, W_mix_e, a_mix,
           W_out):
    lsrc, ldst = lg_edge_index[0], lg_edge_index[1]
    rsrc, rdst = raw_edge_index[0], raw_edge_index[1]

    lsrc_p = jnp.concatenate([lsrc, jnp.zeros((ELG_P - ELG,), i32)])
    ldst_p = jnp.concatenate([ldst, jnp.full((ELG_P - ELG,), E, i32)])
    rsrc_p = jnp.concatenate([rsrc, jnp.zeros((E_P - E,), i32)])
    rdst_p = jnp.concatenate([rdst, jnp.full((E_P - E,), N, i32)])

    s1, s2 = tc_prep(et, W_tsa_in, a_src.reshape(HID, 1), a_dst.reshape(HID, 1))
    s1 = s1.reshape(E)
    s2p = jnp.concatenate([s2.reshape(E), jnp.zeros((8,), f32)])
    ex, d0, d1 = sc_scores(s1, s2p, lsrc_p, ldst_p)
    g_full = sc_gacc(et, lsrc_p, ldst_p, ex)
    tsae = tc_tsae(et, g_full[:E], d0.reshape(E, 1), d1.reshape(E, 1),
                   W_tsa_in, W_tsa_v)

    np0, np1 = sc_nsum(tsae, H)
    cnt_h, cnt_r = sc_counts(H, rdst)
    er = tc_edge_repr(np0, np1, cnt_h.reshape(N, 1), W_etn, W_egcn)

    aggx, agger = sc_agg2(x, er, rsrc_p, rdst_p)
    nh, aer = tc_sage1(x, er, aggx, agger, cnt_r.reshape(N, 1),
                       W_an1_self, W_an1_neigh, W_ea_self, W_ea_neigh)
    ap0, ap1 = sc_agg1(nh, rsrc_p, rdst_p)

    return tc_final(nh, aer, ap0, ap1, cnt_r.reshape(N, 1),
                    W_an2_self, W_an2_neigh, W_mix_n, W_mix_e,
                    a_mix.reshape(2, HID), W_out)


# speedup vs baseline: 12.3943x; 1.0184x over previous
import functools

import jax
import jax.numpy as jnp
from jax import lax
from jax.experimental import pallas as pl
from jax.experimental.pallas import tpu as pltpu, tpu_sc as plsc

N = 10000
E = 320000
ELG = 640000
F = 128
T = 16
HID = 128
OUT = 64

NC = 2
NS = 16
NW = NC * NS

ELG_P = 655360
E_P = 327680
G_P = 327680

CH2 = 2048
EACC = 320256
CH3 = 1024
GROWS = 81920
GACC = 81928
CH5 = 80
CH5B = 2000
NACC1 = 10240
CH7 = 128
NACC2 = 10016

f32 = jnp.float32
i32 = jnp.int32

def _wr_nrows(acc, out, s, nrw):
    @pl.when(nrw == 624)
    def _():
        pltpu.sync_copy(acc.at[pl.ds(s * 624, 624)], out.at[pl.ds(s * 624, 624)])
    @pl.when(nrw == 640)
    def _():
        pltpu.sync_copy(acc.at[pl.ds(s * 624, 640)], out.at[pl.ds(s * 624, 640)])


def _agg_pipeline(tab, rsrc, rdst, acc, base, nch,
                  si, di, r, s_in, s_g):
    CH = CH7

    def it(p, _):
        o0 = base + (p * 2) * CH
        o1 = o0 + CH
        i00 = pltpu.async_copy(rsrc.at[pl.ds(o0, CH)], si[0], s_in[0])
        i01 = pltpu.async_copy(rdst.at[pl.ds(o0, CH)], di[0], s_in[0])
        i10 = pltpu.async_copy(rsrc.at[pl.ds(o1, CH)], si[1], s_in[1])
        i11 = pltpu.async_copy(rdst.at[pl.ds(o1, CH)], di[1], s_in[1])
        i00.wait()
        i01.wait()
        g0 = pltpu.async_copy(tab.at[si[0]], r[0], s_g[0])
        i10.wait()
        i11.wait()
        g1 = pltpu.async_copy(tab.at[si[1]], r[1], s_g[1])
        g0.wait()
        pltpu.sync_copy(r[0], acc.at[di[0]], add=True)
        g1.wait()
        pltpu.sync_copy(r[1], acc.at[di[1]], add=True)
        return 0
    lax.fori_loop(0, nch // 2, it, 0)


_mesh = plsc.VectorSubcoreMesh(core_axis_name="c", subcore_axis_name="s")
_sc_packed = pltpu.CompilerParams(use_tc_tiling_on_sc=False)


@functools.partial(
    pl.kernel,
    out_type=(
        jax.ShapeDtypeStruct((ELG_P,), f32),
        jax.ShapeDtypeStruct((E,), f32),
        jax.ShapeDtypeStruct((E,), f32),
    ),
    mesh=_mesh,
    compiler_params=_sc_packed,
    scratch_types=dict(
        ls_v=pltpu.VMEM((CH2,), i32),
        ld_v=pltpu.VMEM((CH2,), i32),
        g1_v=pltpu.VMEM((CH2,), f32),
        g2_v=pltpu.VMEM((CH2,), f32),
        ex_v=pltpu.VMEM((CH2,), f32),
        z_v=pltpu.VMEM((CH2,), f32),
        acc=pltpu.VMEM_SHARED((EACC,), f32),
        sem=pltpu.SemaphoreType.DMA,
    ),
)
def sc_scores(s1, s2p, lsrc, ldst, ex_out, d0, d1,
              ls_v, ld_v, g1_v, g2_v, ex_v, z_v, acc, sem):
    c = lax.axis_index("c")
    s = lax.axis_index("s")
    wid = s * NC + c

    def zb(i, _):
        z_v[pl.ds(i * 16, 16)] = jnp.zeros((16,), f32)
        return 0
    lax.fori_loop(0, CH2 // 16, zb, 0)
    zbase = s * 20016
    for k in range(9):
        pltpu.sync_copy(z_v, acc.at[pl.ds(zbase + k * CH2, CH2)])
    pltpu.sync_copy(z_v.at[pl.ds(0, 1584)], acc.at[pl.ds(zbase + 9 * CH2, 1584)])
    plsc.subcore_barrier()

    base = wid * (ELG_P // NW)

    def chunk(k, _):
        off = base + k * CH2
        pltpu.sync_copy(lsrc.at[pl.ds(off, CH2)], ls_v)
        pltpu.sync_copy(ldst.at[pl.ds(off, CH2)], ld_v)
        pltpu.async_copy(s1.at[ls_v], g1_v, sem).wait()
        pltpu.async_copy(s2p.at[ld_v], g2_v, sem).wait()

        def grp(g, _):
            v = g1_v[pl.ds(g * 16, 16)] + g2_v[pl.ds(g * 16, 16)]
            v = jnp.where(v >= 0, v, 0.2 * v)
            ex_v[pl.ds(g * 16, 16)] = jnp.exp(v)
            return 0
        lax.fori_loop(0, CH2 // 16, grp, 0)

        pltpu.sync_copy(ex_v, ex_out.at[pl.ds(off, CH2)])
        pltpu.sync_copy(ex_v, acc.at[ld_v], add=True)
        return 0
    lax.fori_loop(0, (ELG_P // NW) // CH2, chunk, 0)
    plsc.subcore_barrier()

    wbase = s * (E // NS)
    @pl.when(c == 0)
    def _():
        pltpu.sync_copy(acc.at[pl.ds(wbase, E // NS)], d0.at[pl.ds(wbase, E // NS)])
    @pl.when(c == 1)
    def _():
        pltpu.sync_copy(acc.at[pl.ds(wbase, E // NS)], d1.at[pl.ds(wbase, E // NS)])


@functools.partial(
    pl.kernel,
    out_type=jax.ShapeDtypeStruct((G_P, T), f32),
    mesh=_mesh,
    compiler_params=_sc_packed,
    scratch_types=dict(
        ls0=pltpu.VMEM((CH3,), i32), ls1=pltpu.VMEM((CH3,), i32),
        ld0=pltpu.VMEM((CH3,), i32), ld1=pltpu.VMEM((CH3,), i32),
        li0=pltpu.VMEM((CH3,), i32), li1=pltpu.VMEM((CH3,), i32),
        ex0=pltpu.VMEM((CH3,), f32), ex1=pltpu.VMEM((CH3,), f32),
        s0=pltpu.VMEM((CH3, T), f32), s1=pltpu.VMEM((CH3, T), f32),
        z_v=pltpu.VMEM((256, T), f32),
        acc=pltpu.VMEM_SHARED((GACC, T), f32),
        m_ls0=pltpu.SemaphoreType.DMA, m_ls1=pltpu.SemaphoreType.DMA,
        m_ld0=pltpu.SemaphoreType.DMA, m_ld1=pltpu.SemaphoreType.DMA,
        m_g0=pltpu.SemaphoreType.DMA, m_g1=pltpu.SemaphoreType.DMA,
    ),
)
def sc_gacc(et, lsrc, ldst, ex, g_out,
            ls0, ls1, ld0, ld1, li0, li1, ex0, ex1, s0, s1, z_v, acc,
            m_ls0, m_ls1, m_ld0, m_ld1, m_g0, m_g1):
    c = lax.axis_index("c")
    s = lax.axis_index("s")
    ls = (ls0, ls1)
    ld = (ld0, ld1)
    li = (li0, li1)
    exb = (ex0, ex1)
    sv = (s0, s1)
    m_ls = (m_ls0, m_ls1)
    m_ld = (m_ld0, m_ld1)
    m_g = (m_g0, m_g1)

    def zrow(i, _):
        z_v[i, :] = jnp.zeros((T,), f32)
        return 0
    lax.fori_loop(0, 256, zrow, 0)

    base = s * (ELG_P // NS)
    nch = (ELG_P // NS) // CH3

    def compute(lo, hi, b):
        def grp(g, _):
            ldg = ld[b][pl.ds(g * 16, 16)]
            inr = (ldg >= lo) & (ldg < hi)
            li[b][pl.ds(g * 16, 16)] = jnp.where(inr, ldg - lo, GROWS)
            exg = exb[b][pl.ds(g * 16, 16)]
            for j in range(16):
                row = g * 16 + j
                sv[b][row, :] = sv[b][row, :] * exg[j]
            return 0
        lax.fori_loop(0, CH3 // 16, grp, 0)

    for r in range(2):
        lo = jnp.where(c == 0, r * 2 * GROWS, (r * 2 + 1) * GROWS).astype(i32)
        hi = lo + GROWS
        zb = s * (GROWS // NS)
        for k in range(GROWS // NS // 256):
            pltpu.sync_copy(z_v, acc.at[pl.ds(zb + k * 256, 256)])
        plsc.subcore_barrier()

        def it(p, _):
            offs = [base + (p * 2 + b) * CH3 for b in range(2)]
            ins = []
            for b in range(2):
                ins.append((
                    pltpu.async_copy(lsrc.at[pl.ds(offs[b], CH3)], ls[b], m_ls[b]),
                    pltpu.async_copy(ldst.at[pl.ds(offs[b], CH3)], ld[b], m_ld[b]),
                    pltpu.async_copy(ex.at[pl.ds(offs[b], CH3)], exb[b], m_ld[b]),
                ))
            gs = []
            for b in range(2):
                ins[b][0].wait()
                gs.append(pltpu.async_copy(et.at[ls[b]], sv[b], m_g[b]))
            for b in range(2):
                gs[b].wait()
                ins[b][1].wait()
                ins[b][2].wait()
                compute(lo, hi, b)
                pltpu.sync_copy(sv[b], acc.at[li[b]], add=True)
            return 0
        lax.fori_loop(0, nch // 2, it, 0)
        plsc.subcore_barrier()

        rps = GROWS // NS
        pltpu.sync_copy(acc.at[pl.ds(s * rps, rps)],
                        g_out.at[pl.ds(lo + s * rps, rps)])
        plsc.subcore_barrier()


@functools.partial(
    pl.kernel,
    out_type=(
        jax.ShapeDtypeStruct((N, HID), f32),
        jax.ShapeDtypeStruct((N, HID), f32),
    ),
    mesh=_mesh,
    scratch_types=dict(
        h_v=pltpu.VMEM((CH5,), i32),
        t_v=pltpu.VMEM((CH5, HID), f32),
        z_v=pltpu.VMEM((64, HID), f32),
        acc=pltpu.VMEM_SHARED((N, HID), f32),
        sem=pltpu.SemaphoreType.DMA,
    ),
)
def sc_nsum(tsae, h_idx, p0, p1, h_v, t_v, z_v, acc, sem):
    c = lax.axis_index("c")
    s = lax.axis_index("s")
    wid = s * NC + c

    def zrow(i, _):
        for j in range(HID // 16):
            z_v[i, pl.ds(j * 16, 16)] = jnp.zeros((16,), f32)
        return 0
    lax.fori_loop(0, 64, zrow, 0)
    zb = s * (N // NS)
    for k in range(9):
        pltpu.sync_copy(z_v, acc.at[pl.ds(zb + k * 64, 64)])
    pltpu.sync_copy(z_v.at[pl.ds(0, 49)], acc.at[pl.ds(zb + 576, 49)])
    plsc.subcore_barrier()

    base = wid * (E // NW)

    def chunk(k, _):
        off = base + k * CH5
        pltpu.sync_copy(h_idx.at[pl.ds(off, CH5)], h_v)
        pltpu.sync_copy(tsae.at[pl.ds(off, CH5), :], t_v)
        pltpu.sync_copy(t_v, acc.at[h_v], add=True)
        return 0
    lax.fori_loop(0, (E // NW) // CH5, chunk, 0)
    plsc.subcore_barrier()

    nrw = jnp.where(s == NS - 1, 640, 624).astype(i32)
    @pl.when(c == 0)
    def _():
        _wr_nrows(acc, p0, s, nrw)
    @pl.when(c == 1)
    def _():
        _wr_nrows(acc, p1, s, nrw)


@functools.partial(
    pl.kernel,
    out_type=(
        jax.ShapeDtypeStruct((N,), f32),
        jax.ShapeDtypeStruct((N,), f32),
    ),
    mesh=_mesh,
    compiler_params=_sc_packed,
    scratch_types=dict(
        i_v=pltpu.VMEM((CH5B,), i32),
        one_v=pltpu.VMEM((CH5B,), f32),
        z_v=pltpu.VMEM((640,), f32),
        acc=pltpu.VMEM_SHARED((NACC1,), f32),
        sem=pltpu.SemaphoreType.DMA,
    ),
)
def sc_counts(h_idx, rdst, cnt_h, cnt_r, i_v, one_v, z_v, acc, sem):
    c = lax.axis_index("c")
    s = lax.axis_index("s")

    def ob(i, _):
        one_v[pl.ds(i * 16, 16)] = jnp.ones((16,), f32)
        return 0
    lax.fori_loop(0, CH5B // 16, ob, 0)
    def zb(i, _):
        z_v[pl.ds(i * 16, 16)] = jnp.zeros((16,), f32)
        return 0
    lax.fori_loop(0, 40, zb, 0)
    pltpu.sync_copy(z_v, acc.at[pl.ds(s * 640, 640)])
    plsc.subcore_barrier()

    base = s * (E // NS)

    def chunk_src(src):
        def chunk(k, _):
            off = base + k * CH5B
            pltpu.sync_copy(src.at[pl.ds(off, CH5B)], i_v)
            pltpu.sync_copy(one_v, acc.at[i_v], add=True)
            return 0
        return chunk

    @pl.when(c == 0)
    def _():
        lax.fori_loop(0, (E // NS) // CH5B, chunk_src(h_idx), 0)
    @pl.when(c == 1)
    def _():
        lax.fori_loop(0, (E // NS) // CH5B, chunk_src(rdst), 0)
    plsc.subcore_barrier()

    @pl.when(s == 0)
    def _():
        @pl.when(c == 0)
        def _():
            pltpu.sync_copy(acc.at[pl.ds(0, N)], cnt_h)
        @pl.when(c == 1)
        def _():
            pltpu.sync_copy(acc.at[pl.ds(0, N)], cnt_r)


@functools.partial(
    pl.kernel,
    out_type=(
        jax.ShapeDtypeStruct((N, HID), f32),
        jax.ShapeDtypeStruct((N, HID), f32),
    ),
    mesh=_mesh,
    scratch_types=dict(
        si0=pltpu.VMEM((CH7,), i32), si1=pltpu.VMEM((CH7,), i32),
        di0=pltpu.VMEM((CH7,), i32), di1=pltpu.VMEM((CH7,), i32),
        r0=pltpu.VMEM((CH7, HID), f32), r1=pltpu.VMEM((CH7, HID), f32),
        z_v=pltpu.VMEM((64, HID), f32),
        acc=pltpu.VMEM_SHARED((NACC2, HID), f32),
        m_i0=pltpu.SemaphoreType.DMA, m_i1=pltpu.SemaphoreType.DMA,
        m_g0=pltpu.SemaphoreType.DMA, m_g1=pltpu.SemaphoreType.DMA,
    ),
)
def sc_agg2(xt, ert, rsrc, rdst, aggx, agger, si0, si1, di0, di1, r0, r1, z_v,
            acc, m_i0, m_i1, m_g0, m_g1):
    c = lax.axis_index("c")
    s = lax.axis_index("s")

    def zrow(i, _):
        for j in range(HID // 16):
            z_v[i, pl.ds(j * 16, 16)] = jnp.zeros((16,), f32)
        return 0
    lax.fori_loop(0, 64, zrow, 0)
    zb = s * (NACC2 // NS)
    for k in range(9):
        pltpu.sync_copy(z_v, acc.at[pl.ds(zb + k * 64, 64)])
    pltpu.sync_copy(z_v.at[pl.ds(0, 50)], acc.at[pl.ds(zb + 576, 50)])
    plsc.subcore_barrier()

    base = s * (E_P // NS)
    nch = (E_P // NS) // CH7

    @pl.when(c == 0)
    def _():
        _agg_pipeline(xt, rsrc, rdst, acc, base, nch,
                      (si0, si1), (di0, di1), (r0, r1),
                      (m_i0, m_i1), (m_g0, m_g1))
    @pl.when(c == 1)
    def _():
        _agg_pipeline(ert, rsrc, rdst, acc, base, nch,
                      (si0, si1), (di0, di1), (r0, r1),
                      (m_i0, m_i1), (m_g0, m_g1))
    plsc.subcore_barrier()

    nrw = jnp.where(s == NS - 1, 640, 624).astype(i32)
    @pl.when(c == 0)
    def _():
        _wr_nrows(acc, aggx, s, nrw)
    @pl.when(c == 1)
    def _():
        _wr_nrows(acc, agger, s, nrw)


@functools.partial(
    pl.kernel,
    out_type=(
        jax.ShapeDtypeStruct((N, HID), f32),
        jax.ShapeDtypeStruct((N, HID), f32),
    ),
    mesh=_mesh,
    scratch_types=dict(
        si0=pltpu.VMEM((CH7,), i32), si1=pltpu.VMEM((CH7,), i32),
        di0=pltpu.VMEM((CH7,), i32), di1=pltpu.VMEM((CH7,), i32),
        r0=pltpu.VMEM((CH7, HID), f32), r1=pltpu.VMEM((CH7, HID), f32),
        z_v=pltpu.VMEM((64, HID), f32),
        acc=pltpu.VMEM_SHARED((NACC2, HID), f32),
        m_i0=pltpu.SemaphoreType.DMA, m_i1=pltpu.SemaphoreType.DMA,
        m_g0=pltpu.SemaphoreType.DMA, m_g1=pltpu.SemaphoreType.DMA,
    ),
)
def sc_agg1(tab, rsrc, rdst, p0, p1, si0, si1, di0, di1, r0, r1, z_v,
            acc, m_i0, m_i1, m_g0, m_g1):
    c = lax.axis_index("c")
    s = lax.axis_index("s")
    wid = s * NC + c

    def zrow(i, _):
        for j in range(HID // 16):
            z_v[i, pl.ds(j * 16, 16)] = jnp.zeros((16,), f32)
        return 0
    lax.fori_loop(0, 64, zrow, 0)
    zb = s * (NACC2 // NS)
    for k in range(9):
        pltpu.sync_copy(z_v, acc.at[pl.ds(zb + k * 64, 64)])
    pltpu.sync_copy(z_v.at[pl.ds(0, 50)], acc.at[pl.ds(zb + 576, 50)])
    plsc.subcore_barrier()

    base = wid * (E_P // NW)
    nch = (E_P // NW) // CH7
    _agg_pipeline(tab, rsrc, rdst, acc, base, nch,
                  (si0, si1), (di0, di1), (r0, r1),
                  (m_i0, m_i1), (m_g0, m_g1))
    plsc.subcore_barrier()

    nrw = jnp.where(s == NS - 1, 640, 624).astype(i32)
    @pl.when(c == 0)
    def _():
        _wr_nrows(acc, p0, s, nrw)
    @pl.when(c == 1)
    def _():
        _wr_nrows(acc, p1, s, nrw)


BLK_E = 3200
BLK_N = 1000


def _tc_prep_body(et_ref, w_ref, a1_ref, a2_ref, s1_ref, s2_ref):
    h = jnp.dot(et_ref[...], w_ref[...], preferred_element_type=f32)
    s1_ref[...] = jnp.dot(h, a1_ref[...], preferred_element_type=f32)
    s2_ref[...] = jnp.dot(h, a2_ref[...], preferred_element_type=f32)


def tc_prep(et, w_tsa_in, a_src, a_dst):
    nb = E // BLK_E
    return pl.pallas_call(
        _tc_prep_body,
        grid=(nb,),
        in_specs=[
            pl.BlockSpec((BLK_E, T), lambda i: (i, 0)),
            pl.BlockSpec((T, HID), lambda i: (0, 0)),
            pl.BlockSpec((HID, 1), lambda i: (0, 0)),
            pl.BlockSpec((HID, 1), lambda i: (0, 0)),
        ],
        out_specs=[
            pl.BlockSpec((BLK_E, 1), lambda i: (i, 0)),
            pl.BlockSpec((BLK_E, 1), lambda i: (i, 0)),
        ],
        out_shape=[
            jax.ShapeDtypeStruct((E, 1), f32),
            jax.ShapeDtypeStruct((E, 1), f32),
        ],
    )(et, w_tsa_in, a_src, a_dst)


def _tc_tsae_body(et_ref, g_ref, d0_ref, d1_ref, w1_ref, wv_ref, o_ref):
    wc = jnp.dot(w1_ref[...], wv_ref[...], preferred_element_type=f32)
    inv = 1.0 / (d0_ref[...] + d1_ref[...] + 1e-16)
    h = jnp.dot(et_ref[...], w1_ref[...], preferred_element_type=f32)
    agg = jnp.dot(g_ref[...] * inv, wc, preferred_element_type=f32)
    o_ref[...] = jnp.maximum(h + agg, 0.0)


def tc_tsae(et, g, d0, d1, w_tsa_in, w_tsa_v):
    nb = E // BLK_E
    return pl.pallas_call(
        _tc_tsae_body,
        grid=(nb,),
        in_specs=[
            pl.BlockSpec((BLK_E, T), lambda i: (i, 0)),
            pl.BlockSpec((BLK_E, T), lambda i: (i, 0)),
            pl.BlockSpec((BLK_E, 1), lambda i: (i, 0)),
            pl.BlockSpec((BLK_E, 1), lambda i: (i, 0)),
            pl.BlockSpec((T, HID), lambda i: (0, 0)),
            pl.BlockSpec((HID, HID), lambda i: (0, 0)),
        ],
        out_specs=pl.BlockSpec((BLK_E, HID), lambda i: (i, 0)),
        out_shape=jax.ShapeDtypeStruct((E, HID), f32),
    )(et, g, d0, d1, w_tsa_in, w_tsa_v)


def _tc_er_body(p0_ref, p1_ref, cnt_ref, wet_ref, weg_ref, o_ref):
    inv = 1.0 / jnp.maximum(cnt_ref[...], 1.0)
    mean = (p0_ref[...] + p1_ref[...]) * inv
    etn = jnp.dot(mean, wet_ref[...], preferred_element_type=f32)
    lre = jnp.where(etn >= 0, etn, 0.2 * etn)
    o_ref[...] = jnp.dot(lre, weg_ref[...], preferred_element_type=f32)


def tc_edge_repr(p0, p1, cnt_h, w_etn, w_egcn):
    nb = N // BLK_N
    return pl.pallas_call(
        _tc_er_body,
        grid=(nb,),
        in_specs=[
            pl.BlockSpec((BLK_N, HID), lambda i: (i, 0)),
            pl.BlockSpec((BLK_N, HID), lambda i: (i, 0)),
            pl.BlockSpec((BLK_N, 1), lambda i: (i, 0)),
            pl.BlockSpec((HID, HID), lambda i: (0, 0)),
            pl.BlockSpec((HID, HID), lambda i: (0, 0)),
        ],
        out_specs=pl.BlockSpec((BLK_N, HID), lambda i: (i, 0)),
        out_shape=jax.ShapeDtypeStruct((N, HID), f32),
    )(p0, p1, cnt_h, w_etn, w_egcn)


def _tc_sage1_body(x_ref, er_ref, ax_ref, ae_ref, cnt_ref,
                   ws1_ref, wn1_ref, wes_ref, wen_ref, nh_ref, aer_ref):
    inv = 1.0 / jnp.maximum(cnt_ref[...], 1.0)
    nh = (jnp.dot(x_ref[...], ws1_ref[...], preferred_element_type=f32)
          + jnp.dot(ax_ref[...] * inv, wn1_ref[...], preferred_element_type=f32))
    nh_ref[...] = jnp.maximum(nh, 0.0)
    aer_ref[...] = (jnp.dot(er_ref[...], wes_ref[...], preferred_element_type=f32)
                    + jnp.dot(ae_ref[...] * inv, wen_ref[...], preferred_element_type=f32))


def tc_sage1(x, er, aggx, agger, cnt_r, w_an1s, w_an1n, w_eas, w_ean):
    nb = N // BLK_N
    return pl.pallas_call(
        _tc_sage1_body,
        grid=(nb,),
        in_specs=[
            pl.BlockSpec((BLK_N, F), lambda i: (i, 0)),
            pl.BlockSpec((BLK_N, HID), lambda i: (i, 0)),
            pl.BlockSpec((BLK_N, F), lambda i: (i, 0)),
            pl.BlockSpec((BLK_N, HID), lambda i: (i, 0)),
            pl.BlockSpec((BLK_N, 1), lambda i: (i, 0)),
            pl.BlockSpec((F, HID), lambda i: (0, 0)),
            pl.BlockSpec((F, HID), lambda i: (0, 0)),
            pl.BlockSpec((HID, HID), lambda i: (0, 0)),
            pl.BlockSpec((HID, HID), lambda i: (0, 0)),
        ],
        out_specs=[
            pl.BlockSpec((BLK_N, HID), lambda i: (i, 0)),
            pl.BlockSpec((BLK_N, HID), lambda i: (i, 0)),
        ],
        out_shape=[
            jax.ShapeDtypeStruct((N, HID), f32),
            jax.ShapeDtypeStruct((N, HID), f32),
        ],
    )(x, er, aggx, agger, cnt_r, w_an1s, w_an1n, w_eas, w_ean)


def _tc_final_body(nh_ref, aer_ref, p0_ref, p1_ref, cnt_ref,
                   w2s_ref, w2n_ref, wmn_ref, wme_ref, am_ref, wo_ref, o_ref):
    inv = 1.0 / jnp.maximum(cnt_ref[...], 1.0)
    nr = (jnp.dot(nh_ref[...], w2s_ref[...], preferred_element_type=f32)
          + jnp.dot((p0_ref[...] + p1_ref[...]) * inv, w2n_ref[...],
                    preferred_element_type=f32))
    zn = jnp.dot(nr, wmn_ref[...], preferred_element_type=f32)
    ze = jnp.dot(aer_ref[...], wme_ref[...], preferred_element_type=f32)
    am = am_ref[...]
    gs = (jnp.sum(zn * am[0:1, :], axis=1, keepdims=True)
          + jnp.sum(ze * am[1:2, :], axis=1, keepdims=True))
    gate = jax.nn.sigmoid(gs)
    mixed = gate * zn + (1.0 - gate) * ze
    logits = jnp.dot(mixed, wo_ref[...], preferred_element_type=f32)
    mx = jnp.max(logits, axis=1, keepdims=True)
    lse = mx + jnp.log(jnp.sum(jnp.exp(logits - mx), axis=1, keepdims=True))
    o_ref[...] = logits - lse


def tc_final(nh, aer, p0, p1, cnt_r, w2s, w2n, wmn, wme, am2, wo):
    nb = N // BLK_N
    return pl.pallas_call(
        _tc_final_body,
        grid=(nb,),
        in_specs=[
            pl.BlockSpec((BLK_N, HID), lambda i: (i, 0)),
            pl.BlockSpec((BLK_N, HID), lambda i: (i, 0)),
            pl.BlockSpec((BLK_N, HID), lambda i: (i, 0)),
            pl.BlockSpec((BLK_N, HID), lambda i: (i, 0)),
            pl.BlockSpec((BLK_N, 1), lambda i: (i, 0)),
            pl.BlockSpec((HID, HID), lambda i: (0, 0)),
            pl.BlockSpec((HID, HID), lambda i: (0, 0)),
            pl.BlockSpec((HID, HID), lambda i: (0, 0)),
            pl.BlockSpec((HID, HID), lambda i: (0, 0)),
            pl.BlockSpec((2, HID), lambda i: (0, 0)),
            pl.BlockSpec((HID, OUT), lambda i: (0, 0)),
        ],
        out_specs=pl.BlockSpec((BLK_N, OUT), lambda i: (i, 0)),
        out_shape=jax.ShapeDtypeStruct((N, OUT), f32),
    )(nh, aer, p0, p1, cnt_r, w2s, w2n, wmn, wme, am2, wo)


def kernel(x, et, H, raw_edge_index, lg_edge_index, W_tsa_in, a_src, a_dst,
           W_tsa_v, W_etn, W_egcn, W_ea_self, W_ea_neigh, W_an1_self,
           W_an1_neigh, W_an2_self, W_an2_neigh, W_mix_n, W_mix_e, a_mix,
           W_out):
    lsrc, ldst = lg_edge_index[0], lg_edge_index[1]
    rsrc, rdst = raw_edge_index[0], raw_edge_index[1]

    lsrc_p = jnp.concatenate([lsrc, jnp.zeros((ELG_P - ELG,), i32)])
    ldst_p = jnp.concatenate([ldst, jnp.full((ELG_P - ELG,), E, i32)])
    rsrc_p = jnp.concatenate([rsrc, jnp.zeros((E_P - E,), i32)])
    rdst_p = jnp.concatenate([rdst, jnp.full((E_P - E,), N, i32)])

    s1, s2 = tc_prep(et, W_tsa_in, a_src.reshape(HID, 1), a_dst.reshape(HID, 1))
    s1 = s1.reshape(E)
    s2p = jnp.concatenate([s2.reshape(E), jnp.zeros((8,), f32)])
    ex, d0, d1 = sc_scores(s1, s2p, lsrc_p, ldst_p)
    g_full = sc_gacc(et, lsrc_p, ldst_p, ex)
    tsae = tc_tsae(et, g_full[:E], d0.reshape(E, 1), d1.reshape(E, 1),
                   W_tsa_in, W_tsa_v)

    np0, np1 = sc_nsum(tsae, H)
    cnt_h, cnt_r = sc_counts(H, rdst)
    er = tc_edge_repr(np0, np1, cnt_h.reshape(N, 1), W_etn, W_egcn)

    aggx, agger = sc_agg2(x, er, rsrc_p, rdst_p)
    nh, aer = tc_sage1(x, er, aggx, agger, cnt_r.reshape(N, 1),
                       W_an1_self, W_an1_neigh, W_ea_self, W_ea_neigh)
    ap0, ap1 = sc_agg1(nh, rsrc_p, rdst_p)

    return tc_final(nh, aer, ap0, ap1, cnt_r.reshape(N, 1),
                    W_an2_self, W_an2_neigh, W_mix_n, W_mix_e,
                    a_mix.reshape(2, HID), W_out)


# speedup vs baseline: 12.7920x; 1.0321x over previous
import functools

import jax
import jax.numpy as jnp
from jax import lax
from jax.experimental import pallas as pl
from jax.experimental.pallas import tpu as pltpu, tpu_sc as plsc

N = 10000
E = 320000
ELG = 640000
F = 128
T = 16
HID = 128
OUT = 64

NC = 2
NS = 16
NW = NC * NS

ELG_P = 655360
E_P = 327680
G_P = 327680

CH2 = 2048
EACC = 320256
CH3 = 1024
GROWS = 81920
GACC = 81928
CH5 = 80
CH5B = 2000
NACC1 = 10240
CH7 = 160
NACC2 = 10016

f32 = jnp.float32
i32 = jnp.int32

def _wr_nrows(acc, out, s, nrw):
    @pl.when(nrw == 624)
    def _():
        pltpu.sync_copy(acc.at[pl.ds(s * 624, 624)], out.at[pl.ds(s * 624, 624)])
    @pl.when(nrw == 640)
    def _():
        pltpu.sync_copy(acc.at[pl.ds(s * 624, 640)], out.at[pl.ds(s * 624, 640)])


def _agg_pipeline(tab, rsrc, rdst, acc, base, nch,
                  si, di, r, s_in, s_g):
    CH = CH7

    def it(p, _):
        o0 = base + (p * 2) * CH
        o1 = o0 + CH
        i00 = pltpu.async_copy(rsrc.at[pl.ds(o0, CH)], si[0], s_in[0])
        i01 = pltpu.async_copy(rdst.at[pl.ds(o0, CH)], di[0], s_in[0])
        i10 = pltpu.async_copy(rsrc.at[pl.ds(o1, CH)], si[1], s_in[1])
        i11 = pltpu.async_copy(rdst.at[pl.ds(o1, CH)], di[1], s_in[1])
        i00.wait()
        i01.wait()
        g0 = pltpu.async_copy(tab.at[si[0]], r[0], s_g[0])
        i10.wait()
        i11.wait()
        g1 = pltpu.async_copy(tab.at[si[1]], r[1], s_g[1])
        g0.wait()
        pltpu.sync_copy(r[0], acc.at[di[0]], add=True)
        g1.wait()
        pltpu.sync_copy(r[1], acc.at[di[1]], add=True)
        return 0
    lax.fori_loop(0, nch // 2, it, 0)


_mesh = plsc.VectorSubcoreMesh(core_axis_name="c", subcore_axis_name="s")
_sc_packed = pltpu.CompilerParams(use_tc_tiling_on_sc=False)


@functools.partial(
    pl.kernel,
    out_type=(
        jax.ShapeDtypeStruct((ELG_P,), f32),
        jax.ShapeDtypeStruct((E,), f32),
        jax.ShapeDtypeStruct((E,), f32),
    ),
    mesh=_mesh,
    compiler_params=_sc_packed,
    scratch_types=dict(
        ls_v=pltpu.VMEM((CH2,), i32),
        ld_v=pltpu.VMEM((CH2,), i32),
        g1_v=pltpu.VMEM((CH2,), f32),
        g2_v=pltpu.VMEM((CH2,), f32),
        ex_v=pltpu.VMEM((CH2,), f32),
        z_v=pltpu.VMEM((CH2,), f32),
        acc=pltpu.VMEM_SHARED((EACC,), f32),
        sem=pltpu.SemaphoreType.DMA,
    ),
)
def sc_scores(s1, s2p, lsrc, ldst, ex_out, d0, d1,
              ls_v, ld_v, g1_v, g2_v, ex_v, z_v, acc, sem):
    c = lax.axis_index("c")
    s = lax.axis_index("s")
    wid = s * NC + c

    def zb(i, _):
        z_v[pl.ds(i * 16, 16)] = jnp.zeros((16,), f32)
        return 0
    lax.fori_loop(0, CH2 // 16, zb, 0)
    zbase = s * 20016
    for k in range(9):
        pltpu.sync_copy(z_v, acc.at[pl.ds(zbase + k * CH2, CH2)])
    pltpu.sync_copy(z_v.at[pl.ds(0, 1584)], acc.at[pl.ds(zbase + 9 * CH2, 1584)])
    plsc.subcore_barrier()

    base = wid * (ELG_P // NW)

    def chunk(k, _):
        off = base + k * CH2
        pltpu.sync_copy(lsrc.at[pl.ds(off, CH2)], ls_v)
        pltpu.sync_copy(ldst.at[pl.ds(off, CH2)], ld_v)
        dg1 = pltpu.async_copy(s1.at[ls_v], g1_v, sem)
        dg2 = pltpu.async_copy(s2p.at[ld_v], g2_v, sem)
        dg1.wait()
        dg2.wait()

        def grp(g, _):
            v = g1_v[pl.ds(g * 16, 16)] + g2_v[pl.ds(g * 16, 16)]
            v = jnp.where(v >= 0, v, 0.2 * v)
            ex_v[pl.ds(g * 16, 16)] = jnp.exp(v)
            return 0
        lax.fori_loop(0, CH2 // 16, grp, 0)

        pltpu.sync_copy(ex_v, ex_out.at[pl.ds(off, CH2)])
        pltpu.sync_copy(ex_v, acc.at[ld_v], add=True)
        return 0
    lax.fori_loop(0, (ELG_P // NW) // CH2, chunk, 0)
    plsc.subcore_barrier()

    wbase = s * (E // NS)
    @pl.when(c == 0)
    def _():
        pltpu.sync_copy(acc.at[pl.ds(wbase, E // NS)], d0.at[pl.ds(wbase, E // NS)])
    @pl.when(c == 1)
    def _():
        pltpu.sync_copy(acc.at[pl.ds(wbase, E // NS)], d1.at[pl.ds(wbase, E // NS)])


@functools.partial(
    pl.kernel,
    out_type=jax.ShapeDtypeStruct((G_P, T), f32),
    mesh=_mesh,
    compiler_params=_sc_packed,
    scratch_types=dict(
        ls0=pltpu.VMEM((CH3,), i32), ls1=pltpu.VMEM((CH3,), i32),
        ld0=pltpu.VMEM((CH3,), i32), ld1=pltpu.VMEM((CH3,), i32),
        li0=pltpu.VMEM((CH3,), i32), li1=pltpu.VMEM((CH3,), i32),
        ex0=pltpu.VMEM((CH3,), f32), ex1=pltpu.VMEM((CH3,), f32),
        s0=pltpu.VMEM((CH3, T), f32), s1=pltpu.VMEM((CH3, T), f32),
        z_v=pltpu.VMEM((256, T), f32),
        acc=pltpu.VMEM_SHARED((GACC, T), f32),
        m_ls0=pltpu.SemaphoreType.DMA, m_ls1=pltpu.SemaphoreType.DMA,
        m_ld0=pltpu.SemaphoreType.DMA, m_ld1=pltpu.SemaphoreType.DMA,
        m_g0=pltpu.SemaphoreType.DMA, m_g1=pltpu.SemaphoreType.DMA,
    ),
)
def sc_gacc(et, lsrc, ldst, ex, g_out,
            ls0, ls1, ld0, ld1, li0, li1, ex0, ex1, s0, s1, z_v, acc,
            m_ls0, m_ls1, m_ld0, m_ld1, m_g0, m_g1):
    c = lax.axis_index("c")
    s = lax.axis_index("s")
    ls = (ls0, ls1)
    ld = (ld0, ld1)
    li = (li0, li1)
    exb = (ex0, ex1)
    sv = (s0, s1)
    m_ls = (m_ls0, m_ls1)
    m_ld = (m_ld0, m_ld1)
    m_g = (m_g0, m_g1)

    def zrow(i, _):
        z_v[i, :] = jnp.zeros((T,), f32)
        return 0
    lax.fori_loop(0, 256, zrow, 0)

    base = s * (ELG_P // NS)
    nch = (ELG_P // NS) // CH3

    def compute(lo, hi, b):
        def grp(g, _):
            ldg = ld[b][pl.ds(g * 16, 16)]
            inr = (ldg >= lo) & (ldg < hi)
            li[b][pl.ds(g * 16, 16)] = jnp.where(inr, ldg - lo, GROWS)
            exg = exb[b][pl.ds(g * 16, 16)]
            for j in range(16):
                row = g * 16 + j
                sv[b][row, :] = sv[b][row, :] * exg[j]
            return 0
        lax.fori_loop(0, CH3 // 16, grp, 0)

    for r in range(2):
        lo = jnp.where(c == 0, r * 2 * GROWS, (r * 2 + 1) * GROWS).astype(i32)
        hi = lo + GROWS
        zb = s * (GROWS // NS)
        for k in range(GROWS // NS // 256):
            pltpu.sync_copy(z_v, acc.at[pl.ds(zb + k * 256, 256)])
        plsc.subcore_barrier()

        def it(p, _):
            offs = [base + (p * 2 + b) * CH3 for b in range(2)]
            ins = []
            for b in range(2):
                ins.append((
                    pltpu.async_copy(lsrc.at[pl.ds(offs[b], CH3)], ls[b], m_ls[b]),
                    pltpu.async_copy(ldst.at[pl.ds(offs[b], CH3)], ld[b], m_ld[b]),
                    pltpu.async_copy(ex.at[pl.ds(offs[b], CH3)], exb[b], m_ld[b]),
                ))
            gs = []
            for b in range(2):
                ins[b][0].wait()
                gs.append(pltpu.async_copy(et.at[ls[b]], sv[b], m_g[b]))
            for b in range(2):
                gs[b].wait()
                ins[b][1].wait()
                ins[b][2].wait()
                compute(lo, hi, b)
                pltpu.sync_copy(sv[b], acc.at[li[b]], add=True)
            return 0
        lax.fori_loop(0, nch // 2, it, 0)
        plsc.subcore_barrier()

        rps = GROWS // NS
        pltpu.sync_copy(acc.at[pl.ds(s * rps, rps)],
                        g_out.at[pl.ds(lo + s * rps, rps)])
        plsc.subcore_barrier()


@functools.partial(
    pl.kernel,
    out_type=(
        jax.ShapeDtypeStruct((N, HID), f32),
        jax.ShapeDtypeStruct((N, HID), f32),
    ),
    mesh=_mesh,
    scratch_types=dict(
        h0=pltpu.VMEM((CH5,), i32), h1=pltpu.VMEM((CH5,), i32),
        t0=pltpu.VMEM((CH5, HID), f32), t1=pltpu.VMEM((CH5, HID), f32),
        z_v=pltpu.VMEM((64, HID), f32),
        acc=pltpu.VMEM_SHARED((N, HID), f32),
        m0=pltpu.SemaphoreType.DMA, m1=pltpu.SemaphoreType.DMA,
    ),
)
def sc_nsum(tsae, h_idx, p0, p1, h0, h1, t0, t1, z_v, acc, m0, m1):
    c = lax.axis_index("c")
    s = lax.axis_index("s")
    wid = s * NC + c

    def zrow(i, _):
        for j in range(HID // 16):
            z_v[i, pl.ds(j * 16, 16)] = jnp.zeros((16,), f32)
        return 0
    lax.fori_loop(0, 64, zrow, 0)
    zb = s * (N // NS)
    for k in range(9):
        pltpu.sync_copy(z_v, acc.at[pl.ds(zb + k * 64, 64)])
    pltpu.sync_copy(z_v.at[pl.ds(0, 49)], acc.at[pl.ds(zb + 576, 49)])
    plsc.subcore_barrier()

    base = wid * (E // NW)
    hb = (h0, h1)
    tb = (t0, t1)
    mb = (m0, m1)

    def it(p, _):
        ds_ = []
        for b in range(2):
            off = base + (p * 2 + b) * CH5
            ds_.append((
                pltpu.async_copy(h_idx.at[pl.ds(off, CH5)], hb[b], mb[b]),
                pltpu.async_copy(tsae.at[pl.ds(off, CH5), :], tb[b], mb[b]),
            ))
        for b in range(2):
            ds_[b][0].wait()
            ds_[b][1].wait()
            pltpu.sync_copy(tb[b], acc.at[hb[b]], add=True)
        return 0
    lax.fori_loop(0, (E // NW) // CH5 // 2, it, 0)
    plsc.subcore_barrier()

    nrw = jnp.where(s == NS - 1, 640, 624).astype(i32)
    @pl.when(c == 0)
    def _():
        _wr_nrows(acc, p0, s, nrw)
    @pl.when(c == 1)
    def _():
        _wr_nrows(acc, p1, s, nrw)


@functools.partial(
    pl.kernel,
    out_type=(
        jax.ShapeDtypeStruct((N,), f32),
        jax.ShapeDtypeStruct((N,), f32),
    ),
    mesh=_mesh,
    compiler_params=_sc_packed,
    scratch_types=dict(
        i_v=pltpu.VMEM((CH5B,), i32),
        one_v=pltpu.VMEM((CH5B,), f32),
        z_v=pltpu.VMEM((640,), f32),
        acc=pltpu.VMEM_SHARED((NACC1,), f32),
        sem=pltpu.SemaphoreType.DMA,
    ),
)
def sc_counts(h_idx, rdst, cnt_h, cnt_r, i_v, one_v, z_v, acc, sem):
    c = lax.axis_index("c")
    s = lax.axis_index("s")

    def ob(i, _):
        one_v[pl.ds(i * 16, 16)] = jnp.ones((16,), f32)
        return 0
    lax.fori_loop(0, CH5B // 16, ob, 0)
    def zb(i, _):
        z_v[pl.ds(i * 16, 16)] = jnp.zeros((16,), f32)
        return 0
    lax.fori_loop(0, 40, zb, 0)
    pltpu.sync_copy(z_v, acc.at[pl.ds(s * 640, 640)])
    plsc.subcore_barrier()

    base = s * (E // NS)

    def chunk_src(src):
        def chunk(k, _):
            off = base + k * CH5B
            pltpu.sync_copy(src.at[pl.ds(off, CH5B)], i_v)
            pltpu.sync_copy(one_v, acc.at[i_v], add=True)
            return 0
        return chunk

    @pl.when(c == 0)
    def _():
        lax.fori_loop(0, (E // NS) // CH5B, chunk_src(h_idx), 0)
    @pl.when(c == 1)
    def _():
        lax.fori_loop(0, (E // NS) // CH5B, chunk_src(rdst), 0)
    plsc.subcore_barrier()

    @pl.when(s == 0)
    def _():
        @pl.when(c == 0)
        def _():
            pltpu.sync_copy(acc.at[pl.ds(0, N)], cnt_h)
        @pl.when(c == 1)
        def _():
            pltpu.sync_copy(acc.at[pl.ds(0, N)], cnt_r)


@functools.partial(
    pl.kernel,
    out_type=(
        jax.ShapeDtypeStruct((N, HID), f32),
        jax.ShapeDtypeStruct((N, HID), f32),
    ),
    mesh=_mesh,
    scratch_types=dict(
        si0=pltpu.VMEM((CH7,), i32), si1=pltpu.VMEM((CH7,), i32),
        di0=pltpu.VMEM((CH7,), i32), di1=pltpu.VMEM((CH7,), i32),
        r0=pltpu.VMEM((CH7, HID), f32), r1=pltpu.VMEM((CH7, HID), f32),
        z_v=pltpu.VMEM((64, HID), f32),
        acc=pltpu.VMEM_SHARED((NACC2, HID), f32),
        m_i0=pltpu.SemaphoreType.DMA, m_i1=pltpu.SemaphoreType.DMA,
        m_g0=pltpu.SemaphoreType.DMA, m_g1=pltpu.SemaphoreType.DMA,
    ),
)
def sc_agg2(xt, ert, rsrc, rdst, aggx, agger, si0, si1, di0, di1, r0, r1, z_v,
            acc, m_i0, m_i1, m_g0, m_g1):
    c = lax.axis_index("c")
    s = lax.axis_index("s")

    def zrow(i, _):
        for j in range(HID // 16):
            z_v[i, pl.ds(j * 16, 16)] = jnp.zeros((16,), f32)
        return 0
    lax.fori_loop(0, 64, zrow, 0)
    zb = s * (NACC2 // NS)
    for k in range(9):
        pltpu.sync_copy(z_v, acc.at[pl.ds(zb + k * 64, 64)])
    pltpu.sync_copy(z_v.at[pl.ds(0, 50)], acc.at[pl.ds(zb + 576, 50)])
    plsc.subcore_barrier()

    base = s * (E_P // NS)
    nch = (E_P // NS) // CH7

    @pl.when(c == 0)
    def _():
        _agg_pipeline(xt, rsrc, rdst, acc, base, nch,
                      (si0, si1), (di0, di1), (r0, r1),
                      (m_i0, m_i1), (m_g0, m_g1))
    @pl.when(c == 1)
    def _():
        _agg_pipeline(ert, rsrc, rdst, acc, base, nch,
                      (si0, si1), (di0, di1), (r0, r1),
                      (m_i0, m_i1), (m_g0, m_g1))
    plsc.subcore_barrier()

    nrw = jnp.where(s == NS - 1, 640, 624).astype(i32)
    @pl.when(c == 0)
    def _():
        _wr_nrows(acc, aggx, s, nrw)
    @pl.when(c == 1)
    def _():
        _wr_nrows(acc, agger, s, nrw)


@functools.partial(
    pl.kernel,
    out_type=(
        jax.ShapeDtypeStruct((N, HID), f32),
        jax.ShapeDtypeStruct((N, HID), f32),
    ),
    mesh=_mesh,
    scratch_types=dict(
        si0=pltpu.VMEM((CH7,), i32), si1=pltpu.VMEM((CH7,), i32),
        di0=pltpu.VMEM((CH7,), i32), di1=pltpu.VMEM((CH7,), i32),
        r0=pltpu.VMEM((CH7, HID), f32), r1=pltpu.VMEM((CH7, HID), f32),
        z_v=pltpu.VMEM((64, HID), f32),
        acc=pltpu.VMEM_SHARED((NACC2, HID), f32),
        m_i0=pltpu.SemaphoreType.DMA, m_i1=pltpu.SemaphoreType.DMA,
        m_g0=pltpu.SemaphoreType.DMA, m_g1=pltpu.SemaphoreType.DMA,
    ),
)
def sc_agg1(tab, rsrc, rdst, p0, p1, si0, si1, di0, di1, r0, r1, z_v,
            acc, m_i0, m_i1, m_g0, m_g1):
    c = lax.axis_index("c")
    s = lax.axis_index("s")
    wid = s * NC + c

    def zrow(i, _):
        for j in range(HID // 16):
            z_v[i, pl.ds(j * 16, 16)] = jnp.zeros((16,), f32)
        return 0
    lax.fori_loop(0, 64, zrow, 0)
    zb = s * (NACC2 // NS)
    for k in range(9):
        pltpu.sync_copy(z_v, acc.at[pl.ds(zb + k * 64, 64)])
    pltpu.sync_copy(z_v.at[pl.ds(0, 50)], acc.at[pl.ds(zb + 576, 50)])
    plsc.subcore_barrier()

    base = wid * (E_P // NW)
    nch = (E_P // NW) // CH7
    _agg_pipeline(tab, rsrc, rdst, acc, base, nch,
                  (si0, si1), (di0, di1), (r0, r1),
                  (m_i0, m_i1), (m_g0, m_g1))
    plsc.subcore_barrier()

    nrw = jnp.where(s == NS - 1, 640, 624).astype(i32)
    @pl.when(c == 0)
    def _():
        _wr_nrows(acc, p0, s, nrw)
    @pl.when(c == 1)
    def _():
        _wr_nrows(acc, p1, s, nrw)


BLK_E = 3200
BLK_N = 1000


def _tc_prep_body(et_ref, w_ref, a1_ref, a2_ref, s1_ref, s2_ref):
    h = jnp.dot(et_ref[...], w_ref[...], preferred_element_type=f32)
    s1_ref[...] = jnp.dot(h, a1_ref[...], preferred_element_type=f32)
    s2_ref[...] = jnp.dot(h, a2_ref[...], preferred_element_type=f32)


def tc_prep(et, w_tsa_in, a_src, a_dst):
    nb = E // BLK_E
    return pl.pallas_call(
        _tc_prep_body,
        grid=(nb,),
        in_specs=[
            pl.BlockSpec((BLK_E, T), lambda i: (i, 0)),
            pl.BlockSpec((T, HID), lambda i: (0, 0)),
            pl.BlockSpec((HID, 1), lambda i: (0, 0)),
            pl.BlockSpec((HID, 1), lambda i: (0, 0)),
        ],
        out_specs=[
            pl.BlockSpec((BLK_E, 1), lambda i: (i, 0)),
            pl.BlockSpec((BLK_E, 1), lambda i: (i, 0)),
        ],
        out_shape=[
            jax.ShapeDtypeStruct((E, 1), f32),
            jax.ShapeDtypeStruct((E, 1), f32),
        ],
    )(et, w_tsa_in, a_src, a_dst)


def _tc_tsae_body(et_ref, g_ref, d0_ref, d1_ref, w1_ref, wv_ref, o_ref):
    wc = jnp.dot(w1_ref[...], wv_ref[...], preferred_element_type=f32)
    inv = 1.0 / (d0_ref[...] + d1_ref[...] + 1e-16)
    h = jnp.dot(et_ref[...], w1_ref[...], preferred_element_type=f32)
    agg = jnp.dot(g_ref[...] * inv, wc, preferred_element_type=f32)
    o_ref[...] = jnp.maximum(h + agg, 0.0)


def tc_tsae(et, g, d0, d1, w_tsa_in, w_tsa_v):
    nb = E // BLK_E
    return pl.pallas_call(
        _tc_tsae_body,
        grid=(nb,),
        in_specs=[
            pl.BlockSpec((BLK_E, T), lambda i: (i, 0)),
            pl.BlockSpec((BLK_E, T), lambda i: (i, 0)),
            pl.BlockSpec((BLK_E, 1), lambda i: (i, 0)),
            pl.BlockSpec((BLK_E, 1), lambda i: (i, 0)),
            pl.BlockSpec((T, HID), lambda i: (0, 0)),
            pl.BlockSpec((HID, HID), lambda i: (0, 0)),
        ],
        out_specs=pl.BlockSpec((BLK_E, HID), lambda i: (i, 0)),
        out_shape=jax.ShapeDtypeStruct((E, HID), f32),
    )(et, g, d0, d1, w_tsa_in, w_tsa_v)


def _tc_er_body(p0_ref, p1_ref, cnt_ref, wet_ref, weg_ref, o_ref):
    inv = 1.0 / jnp.maximum(cnt_ref[...], 1.0)
    mean = (p0_ref[...] + p1_ref[...]) * inv
    etn = jnp.dot(mean, wet_ref[...], preferred_element_type=f32)
    lre = jnp.where(etn >= 0, etn, 0.2 * etn)
    o_ref[...] = jnp.dot(lre, weg_ref[...], preferred_element_type=f32)


def tc_edge_repr(p0, p1, cnt_h, w_etn, w_egcn):
    nb = N // BLK_N
    return pl.pallas_call(
        _tc_er_body,
        grid=(nb,),
        in_specs=[
            pl.BlockSpec((BLK_N, HID), lambda i: (i, 0)),
            pl.BlockSpec((BLK_N, HID), lambda i: (i, 0)),
            pl.BlockSpec((BLK_N, 1), lambda i: (i, 0)),
            pl.BlockSpec((HID, HID), lambda i: (0, 0)),
            pl.BlockSpec((HID, HID), lambda i: (0, 0)),
        ],
        out_specs=pl.BlockSpec((BLK_N, HID), lambda i: (i, 0)),
        out_shape=jax.ShapeDtypeStruct((N, HID), f32),
    )(p0, p1, cnt_h, w_etn, w_egcn)


def _tc_sage1_body(x_ref, er_ref, ax_ref, ae_ref, cnt_ref,
                   ws1_ref, wn1_ref, wes_ref, wen_ref, nh_ref, aer_ref):
    inv = 1.0 / jnp.maximum(cnt_ref[...], 1.0)
    nh = (jnp.dot(x_ref[...], ws1_ref[...], preferred_element_type=f32)
          + jnp.dot(ax_ref[...] * inv, wn1_ref[...], preferred_element_type=f32))
    nh_ref[...] = jnp.maximum(nh, 0.0)
    aer_ref[...] = (jnp.dot(er_ref[...], wes_ref[...], preferred_element_type=f32)
                    + jnp.dot(ae_ref[...] * inv, wen_ref[...], preferred_element_type=f32))


def tc_sage1(x, er, aggx, agger, cnt_r, w_an1s, w_an1n, w_eas, w_ean):
    nb = N // BLK_N
    return pl.pallas_call(
        _tc_sage1_body,
        grid=(nb,),
        in_specs=[
            pl.BlockSpec((BLK_N, F), lambda i: (i, 0)),
            pl.BlockSpec((BLK_N, HID), lambda i: (i, 0)),
            pl.BlockSpec((BLK_N, F), lambda i: (i, 0)),
            pl.BlockSpec((BLK_N, HID), lambda i: (i, 0)),
            pl.BlockSpec((BLK_N, 1), lambda i: (i, 0)),
            pl.BlockSpec((F, HID), lambda i: (0, 0)),
            pl.BlockSpec((F, HID), lambda i: (0, 0)),
            pl.BlockSpec((HID, HID), lambda i: (0, 0)),
            pl.BlockSpec((HID, HID), lambda i: (0, 0)),
        ],
        out_specs=[
            pl.BlockSpec((BLK_N, HID), lambda i: (i, 0)),
            pl.BlockSpec((BLK_N, HID), lambda i: (i, 0)),
        ],
        out_shape=[
            jax.ShapeDtypeStruct((N, HID), f32),
            jax.ShapeDtypeStruct((N, HID), f32),
        ],
    )(x, er, aggx, agger, cnt_r, w_an1s, w_an1n, w_eas, w_ean)


def _tc_final_body(nh_ref, aer_ref, p0_ref, p1_ref, cnt_ref,
                   w2s_ref, w2n_ref, wmn_ref, wme_ref, am_ref, wo_ref, o_ref):
    inv = 1.0 / jnp.maximum(cnt_ref[...], 1.0)
    nr = (jnp.dot(nh_ref[...], w2s_ref[...], preferred_element_type=f32)
          + jnp.dot((p0_ref[...] + p1_ref[...]) * inv, w2n_ref[...],
                    preferred_element_type=f32))
    zn = jnp.dot(nr, wmn_ref[...], preferred_element_type=f32)
    ze = jnp.dot(aer_ref[...], wme_ref[...], preferred_element_type=f32)
    am = am_ref[...]
    gs = (jnp.sum(zn * am[0:1, :], axis=1, keepdims=True)
          + jnp.sum(ze * am[1:2, :], axis=1, keepdims=True))
    gate = jax.nn.sigmoid(gs)
    mixed = gate * zn + (1.0 - gate) * ze
    logits = jnp.dot(mixed, wo_ref[...], preferred_element_type=f32)
    mx = jnp.max(logits, axis=1, keepdims=True)
    lse = mx + jnp.log(jnp.sum(jnp.exp(logits - mx), axis=1, keepdims=True))
    o_ref[...] = logits - lse


def tc_final(nh, aer, p0, p1, cnt_r, w2s, w2n, wmn, wme, am2, wo):
    nb = N // BLK_N
    return pl.pallas_call(
        _tc_final_body,
        grid=(nb,),
        in_specs=[
            pl.BlockSpec((BLK_N, HID), lambda i: (i, 0)),
            pl.BlockSpec((BLK_N, HID), lambda i: (i, 0)),
            pl.BlockSpec((BLK_N, HID), lambda i: (i, 0)),
            pl.BlockSpec((BLK_N, HID), lambda i: (i, 0)),
            pl.BlockSpec((BLK_N, 1), lambda i: (i, 0)),
            pl.BlockSpec((HID, HID), lambda i: (0, 0)),
            pl.BlockSpec((HID, HID), lambda i: (0, 0)),
            pl.BlockSpec((HID, HID), lambda i: (0, 0)),
            pl.BlockSpec((HID, HID), lambda i: (0, 0)),
            pl.BlockSpec((2, HID), lambda i: (0, 0)),
            pl.BlockSpec((HID, OUT), lambda i: (0, 0)),
        ],
        out_specs=pl.BlockSpec((BLK_N, OUT), lambda i: (i, 0)),
        out_shape=jax.ShapeDtypeStruct((N, OUT), f32),
    )(nh, aer, p0, p1, cnt_r, w2s, w2n, wmn, wme, am2, wo)


def kernel(x, et, H, raw_edge_index, lg_edge_index, W_tsa_in, a_src, a_dst,
           W_tsa_v, W_etn, W_egcn, W_ea_self, W_ea_neigh, W_an1_self,
           W_an1_neigh, W_an2_self, W_an2_neigh, W_mix_n, W_mix_e, a_mix,
           W_out):
    lsrc, ldst = lg_edge_index[0], lg_edge_index[1]
    rsrc, rdst = raw_edge_index[0], raw_edge_index[1]

    lsrc_p = jnp.concatenate([lsrc, jnp.zeros((ELG_P - ELG,), i32)])
    ldst_p = jnp.concatenate([ldst, jnp.full((ELG_P - ELG,), E, i32)])
    rsrc_p = jnp.concatenate([rsrc, jnp.zeros((E_P - E,), i32)])
    rdst_p = jnp.concatenate([rdst, jnp.full((E_P - E,), N, i32)])

    s1, s2 = tc_prep(et, W_tsa_in, a_src.reshape(HID, 1), a_dst.reshape(HID, 1))
    s1 = s1.reshape(E)
    s2p = jnp.concatenate([s2.reshape(E), jnp.zeros((8,), f32)])
    ex, d0, d1 = sc_scores(s1, s2p, lsrc_p, ldst_p)
    g_full = sc_gacc(et, lsrc_p, ldst_p, ex)
    tsae = tc_tsae(et, g_full[:E], d0.reshape(E, 1), d1.reshape(E, 1),
                   W_tsa_in, W_tsa_v)

    np0, np1 = sc_nsum(tsae, H)
    cnt_h, cnt_r = sc_counts(H, rdst)
    er = tc_edge_repr(np0, np1, cnt_h.reshape(N, 1), W_etn, W_egcn)

    aggx, agger = sc_agg2(x, er, rsrc_p, rdst_p)
    nh, aer = tc_sage1(x, er, aggx, agger, cnt_r.reshape(N, 1),
                       W_an1_self, W_an1_neigh, W_ea_self, W_ea_neigh)
    ap0, ap1 = sc_agg1(nh, rsrc_p, rdst_p)

    return tc_final(nh, aer, ap0, ap1, cnt_r.reshape(N, 1),
                    W_an2_self, W_an2_neigh, W_mix_n, W_mix_e,
                    a_mix.reshape(2, HID), W_out)
